# Initial kernel scaffold; baseline (speedup 1.0000x reference)
import jax
import jax.numpy as jnp
from jax.experimental import pallas as pl


def kernel(x, W, b):
    raise NotImplementedError("write your pallas kernel here")


# speedup vs baseline: 38.1274x; 38.1274x over previous
import jax
import jax.numpy as jnp
from jax.experimental import pallas as pl
from jax.experimental.pallas import tpu as pltpu

LONG, LAT = 500, 200
N = LONG * LAT
I_LO, I_HI = LAT + 1, (LONG - 1) * LAT - 2
OFFS = (-LAT - 1, -LAT, -LAT + 1, -1, 1, LAT - 1, LAT, LAT + 1)
PAD = 256
NCHUNK = 8
CK = 12544
TOT = NCHUNK * CK
XLEN = PAD + TOT + PAD


def _fused_kernel(x_ref, w_ref, b_ref, o_ref, h1_ref):
    k = pl.program_id(0)

    @pl.when(k == 0)
    def _stencil():
        for p in range(NCHUNK):
            base = PAD + p * CK
            j = jax.lax.broadcasted_iota(jnp.int32, (1, CK), 1) + p * CK
            in_i = ((j >= I_LO) & (j <= I_HI)).astype(jnp.float32)
            h1 = x_ref[:, base:base + CK]
            for o in OFFS:
                jo = j + o
                coeff = in_i + ((jo >= I_LO) & (jo <= I_HI)).astype(jnp.float32)
                h1 = h1 + x_ref[:, base + o:base + o + CK] * coeff
            h1_ref[:, p * CK:(p + 1) * CK] = h1

    j = jax.lax.broadcasted_iota(jnp.int32, (1, CK), 1) + k * CK
    w = jnp.where(j < N, w_ref[...], 0.0)
    acc = jax.lax.dot_general(
        h1_ref[:, pl.ds(k * CK, CK)], w,
        (((1,), (1,)), ((), ())),
        preferred_element_type=jnp.float32,
        precision=jax.lax.Precision.HIGHEST,
    )

    @pl.when(k == 0)
    def _init():
        o_ref[...] = acc + b_ref[...]

    @pl.when(k != 0)
    def _acc():
        o_ref[...] += acc


@jax.jit
def kernel(x, W, b):
    B = x.shape[0]
    x_flat = x.reshape(B, N).astype(jnp.float32)
    x_pad = jnp.pad(x_flat, ((0, 0), (PAD, XLEN - N - PAD)))
    b2 = b.reshape(1, 256)
    out = pl.pallas_call(
        _fused_kernel,
        grid=(NCHUNK,),
        in_specs=[
            pl.BlockSpec((B, XLEN), lambda k: (0, 0)),
            pl.BlockSpec((256, CK), lambda k: (0, k)),
            pl.BlockSpec((1, 256), lambda k: (0, 0)),
        ],
        out_specs=pl.BlockSpec((B, 256), lambda k: (0, 0)),
        out_shape=jax.ShapeDtypeStruct((B, 256), jnp.float32),
        scratch_shapes=[pltpu.VMEM((B, TOT), jnp.float32)],
    )(x_pad, W, b2)
    return out


# speedup vs baseline: 49.6833x; 1.3031x over previous
import jax
import jax.numpy as jnp
from jax.experimental import pallas as pl
from jax.experimental.pallas import tpu as pltpu

LONG, LAT = 500, 200
N = LONG * LAT
I_LO, I_HI = LAT + 1, (LONG - 1) * LAT - 2
OFFS = (-LAT - 1, -LAT, -LAT + 1, -1, 1, LAT - 1, LAT, LAT + 1)
PAD = 256
NCHUNK = 8
CK = 12544
TOT = NCHUNK * CK
XLEN = PAD + TOT + PAD


def _fused_kernel(x_ref, w_ref, b_ref, o_ref, h1_ref):
    k = pl.program_id(0)

    @pl.when(k == 0)
    def _stencil():
        for p in range(NCHUNK):
            base = PAD + p * CK
            j = jax.lax.broadcasted_iota(jnp.int32, (1, CK), 1) + p * CK
            in_i = ((j >= I_LO) & (j <= I_HI)).astype(jnp.float32)
            h1 = x_ref[:, base:base + CK]
            for o in OFFS:
                jo = j + o
                coeff = in_i + ((jo >= I_LO) & (jo <= I_HI)).astype(jnp.float32)
                h1 = h1 + x_ref[:, base + o:base + o + CK] * coeff
            h1_ref[:, p * CK:(p + 1) * CK] = h1

    j = jax.lax.broadcasted_iota(jnp.int32, (1, CK), 1) + k * CK
    w = jnp.where(j < N, w_ref[...], 0.0)
    acc = jax.lax.dot_general(
        h1_ref[:, pl.ds(k * CK, CK)], w,
        (((1,), (1,)), ((), ())),
        preferred_element_type=jnp.float32,
        precision=jax.lax.Precision.DEFAULT,
    )

    @pl.when(k == 0)
    def _init():
        o_ref[...] = acc + b_ref[...]

    @pl.when(k != 0)
    def _acc():
        o_ref[...] += acc


@jax.jit
def kernel(x, W, b):
    B = x.shape[0]
    x_flat = x.reshape(B, N).astype(jnp.float32)
    x_pad = jnp.pad(x_flat, ((0, 0), (PAD, XLEN - N - PAD)))
    b2 = b.reshape(1, 256)
    out = pl.pallas_call(
        _fused_kernel,
        grid=(NCHUNK,),
        in_specs=[
            pl.BlockSpec((B, XLEN), lambda k: (0, 0)),
            pl.BlockSpec((256, CK), lambda k: (0, k)),
            pl.BlockSpec((1, 256), lambda k: (0, 0)),
        ],
        out_specs=pl.BlockSpec((B, 256), lambda k: (0, 0)),
        out_shape=jax.ShapeDtypeStruct((B, 256), jnp.float32),
        scratch_shapes=[pltpu.VMEM((B, TOT), jnp.float32)],
    )(x_pad, W, b2)
    return out


# speedup vs baseline: 51.3345x; 1.0332x over previous
import jax
import jax.numpy as jnp
from jax.experimental import pallas as pl
from jax.experimental.pallas import tpu as pltpu

LONG, LAT = 500, 200
N = LONG * LAT
I_LO, I_HI = LAT + 1, (LONG - 1) * LAT - 2
OFFS = (-LAT - 1, -LAT, -LAT + 1, -1, 1, LAT - 1, LAT, LAT + 1)
PAD = 256
NCHUNK = 8
CK = 12544
TOT = NCHUNK * CK
XLEN = PAD + TOT + PAD


def _fused_kernel(x_ref, w_ref, b_ref, o_ref, xp_ref, h1_ref):
    k = pl.program_id(0)

    @pl.when(k == 0)
    def _stencil():
        B = x_ref.shape[0]
        xp_ref[:, :PAD] = jnp.zeros((B, PAD), jnp.float32)
        xp_ref[:, PAD:PAD + N] = x_ref[...]
        xp_ref[:, PAD + N:] = jnp.zeros((B, XLEN - PAD - N), jnp.float32)
        for p in range(NCHUNK):
            base = PAD + p * CK
            j = jax.lax.broadcasted_iota(jnp.int32, (1, CK), 1) + p * CK
            in_i = ((j >= I_LO) & (j <= I_HI)).astype(jnp.float32)
            h1 = xp_ref[:, base:base + CK]
            for o in OFFS:
                jo = j + o
                coeff = in_i + ((jo >= I_LO) & (jo <= I_HI)).astype(jnp.float32)
                h1 = h1 + xp_ref[:, base + o:base + o + CK] * coeff
            h1_ref[:, p * CK:(p + 1) * CK] = h1

    j = jax.lax.broadcasted_iota(jnp.int32, (1, CK), 1) + k * CK
    w = jnp.where(j < N, w_ref[...], 0.0)
    acc = jax.lax.dot_general(
        h1_ref[:, pl.ds(k * CK, CK)], w,
        (((1,), (1,)), ((), ())),
        preferred_element_type=jnp.float32,
        precision=jax.lax.Precision.DEFAULT,
    )

    @pl.when(k == 0)
    def _init():
        o_ref[...] = acc + b_ref[...]

    @pl.when(k != 0)
    def _acc():
        o_ref[...] += acc


@jax.jit
def kernel(x, W, b):
    B = x.shape[0]
    x_flat = x.reshape(B, N).astype(jnp.float32)
    b2 = b.reshape(1, 256)
    out = pl.pallas_call(
        _fused_kernel,
        grid=(NCHUNK,),
        in_specs=[
            pl.BlockSpec((B, N), lambda k: (0, 0)),
            pl.BlockSpec((256, CK), lambda k: (0, k)),
            pl.BlockSpec((1, 256), lambda k: (0, 0)),
        ],
        out_specs=pl.BlockSpec((B, 256), lambda k: (0, 0)),
        out_shape=jax.ShapeDtypeStruct((B, 256), jnp.float32),
        scratch_shapes=[
            pltpu.VMEM((B, XLEN), jnp.float32),
            pltpu.VMEM((B, TOT), jnp.float32),
        ],
    )(x_flat, W, b2)
    return out
